# Initial kernel scaffold; baseline (speedup 1.0000x reference)
import jax
import jax.numpy as jnp
from jax.experimental import pallas as pl


def kernel(features, W_e1, b_e1, g_bn1, bt_bn1, W_e2, b_e2, g_emb, bt_emb, W_lin0, W_src0, W_dst0, W_pos0, b_pos0, g_t0, bt_t0, W_lin1, W_src1, W_dst1, W_pos1, b_pos1, g_t1, bt_t1, W_d1, b_d1, g_d, bt_d, W_d2, b_d2):
    raise NotImplementedError("write your pallas kernel here")


# speedup vs baseline: 7.3809x; 7.3809x over previous
import functools

import jax
import jax.numpy as jnp
from jax import lax
from jax.experimental import pallas as pl
from jax.experimental.pallas import tpu as pltpu
from jax.experimental.pallas import tpu_sc as plsc

B, N, K, NCLS = 4, 5000, 16, 13
ED, HD = 64, 128
NT = B * N
GW = 256
EPS = 1e-5

NSC, NSUB = 2, 16
NW = NSC * NSUB
CH = 8
NCHTOT = NT // CH
NITER = (NCHTOT + NW - 1) // NW
GR = K * CH

RT = 2000
NRT = NT // RT
KT = 40
NKT = N // KT


def _embed_body(f_ref, we1, be1, g1, bt1, we2, be2, g2, bt2, wsrc, wlin, wpos,
                g_ref, accF, sF, accX, sX):
    p_ = pl.program_id(0)
    t_ = pl.program_id(1)
    n = jnp.float32(NT)
    f = f_ref[...]

    def bn_lin(x, W, bvec, g, bt, acc, s):
        mu_x = s[...] / n
        M = acc[...] / n
        ml = jnp.dot(mu_x, W, precision=lax.Precision.HIGHEST, preferred_element_type=jnp.float32)
        MW = jnp.dot(M, W, precision=lax.Precision.HIGHEST, preferred_element_type=jnp.float32)
        var = jnp.sum(MW * W, axis=0, keepdims=True) - ml * ml
        h = jnp.dot(x, W, preferred_element_type=jnp.float32) + bvec
        xo = (h - (ml + bvec)) * lax.rsqrt(var + EPS) * g + bt
        return jnp.maximum(xo, 0.0)

    def x1_of(f):
        return bn_lin(f, we1[...], be1[...], g1[...], bt1[...], accF, sF)

    @pl.when(p_ == 0)
    def _():
        ff = lax.dot_general(f, f, (((0,), (0,)), ((), ())),
                             precision=lax.Precision.HIGHEST, preferred_element_type=jnp.float32)
        sf = jnp.sum(f, axis=0, keepdims=True)
        accF[...] = jnp.where(t_ == 0, ff, accF[...] + ff)
        sF[...] = jnp.where(t_ == 0, sf, sF[...] + sf)

    @pl.when(p_ == 1)
    def _():
        x1 = x1_of(f)
        xx = lax.dot_general(x1, x1, (((0,), (0,)), ((), ())),
                             precision=lax.Precision.HIGHEST, preferred_element_type=jnp.float32)
        sx = jnp.sum(x1, axis=0, keepdims=True)
        accX[...] = jnp.where(t_ == 0, xx, accX[...] + xx)
        sX[...] = jnp.where(t_ == 0, sx, sX[...] + sx)

    @pl.when(p_ == 2)
    def _():
        x1 = x1_of(f)
        x2 = bn_lin(x1, we2[...], be2[...], g2[...], bt2[...], accX, sX)
        a = jnp.dot(x2, wsrc[...], preferred_element_type=jnp.float32)
        v = jnp.dot(x2, wlin[...], preferred_element_type=jnp.float32)
        P = jnp.dot(f[:, 0:3], wpos[...], precision=lax.Precision.HIGHEST, preferred_element_type=jnp.float32)
        g_ref[:, 0:HD] = a + P
        g_ref[:, HD:2 * HD] = v - P


def _embed(feats, we1, be1, g1, bt1, we2, be2, g2, bt2, wsrc, wlin, wpos):
    full = lambda s: pl.BlockSpec(s, lambda p, t: (0, 0))
    return pl.pallas_call(
        _embed_body,
        grid=(3, NRT),
        in_specs=[
            pl.BlockSpec((RT, 6), lambda p, t: (t, 0)),
            full((6, ED)), full((1, ED)), full((1, ED)), full((1, ED)),
            full((ED, ED)), full((1, ED)), full((1, ED)), full((1, ED)),
            full((ED, HD)), full((ED, HD)), full((3, HD)),
        ],
        out_specs=pl.BlockSpec((RT, GW), lambda p, t: (t, 0)),
        out_shape=jax.ShapeDtypeStruct((NT, GW), jnp.float32),
        scratch_shapes=[
            pltpu.VMEM((6, 6), jnp.float32),
            pltpu.VMEM((1, 6), jnp.float32),
            pltpu.VMEM((ED, ED), jnp.float32),
            pltpu.VMEM((1, ED), jnp.float32),
        ],
    )(feats, we1, be1, g1, bt1, we2, be2, g2, bt2, wsrc, wlin, wpos)


def _knn_body(fr_ref, fa_ref, idx_ref):
    b = pl.program_id(0)
    t = pl.program_id(1)
    pr = fr_ref[0, :, 0:3]
    pa = fa_ref[0, :, 0:3]
    sq_r = jnp.sum(pr * pr, axis=1)
    sq_a = jnp.sum(pa * pa, axis=1)
    G = lax.dot_general(pr, pa, (((1,), (1,)), ((), ())),
                        preferred_element_type=jnp.float32)
    d2 = sq_r[:, None] + sq_a[None, :] - 2.0 * G
    col = lax.broadcasted_iota(jnp.int32, (KT, N), 1)
    rowg = t * KT + lax.broadcasted_iota(jnp.int32, (KT, N), 0)
    d2 = jnp.where(col == rowg, 1e30, d2)
    outs = []
    for _ in range(K):
        m = jnp.min(d2, axis=1, keepdims=True)
        sel = jnp.min(jnp.where(d2 == m, col, jnp.int32(2 ** 30)),
                      axis=1, keepdims=True)
        outs.append(sel)
        d2 = jnp.where(col == sel, 1e30, d2)
    idx_ref[0, :, :] = jnp.concatenate(outs, axis=1) + b * N


def _knn(features):
    return pl.pallas_call(
        _knn_body,
        grid=(B, NKT),
        in_specs=[
            pl.BlockSpec((1, KT, 6), lambda b, t: (b, t, 0)),
            pl.BlockSpec((1, N, 6), lambda b, t: (b, 0, 0)),
        ],
        out_specs=pl.BlockSpec((1, KT, K), lambda b, t: (b, t, 0)),
        out_shape=jax.ShapeDtypeStruct((B, N, K), jnp.int32),
    )(features, features)


def _conv(Gm, idxf):
    mesh = plsc.VectorSubcoreMesh(core_axis_name="c", subcore_axis_name="s")

    @functools.partial(
        pl.kernel,
        mesh=mesh,
        out_type=jax.ShapeDtypeStruct((NT, HD), jnp.float32),
        scratch_types=[
            pltpu.VMEM((GR + CH, GW), jnp.float32),
            pltpu.VMEM((GR,), jnp.int32),
            pltpu.VMEM((CH, HD), jnp.float32),
            pltpu.SemaphoreType.DMA,
        ],
    )
    def conv(G_hbm, idx_hbm, out_hbm, buf, ilist, ostage, sem):
        cid = lax.axis_index("c")
        sid = lax.axis_index("s")
        wid = sid * NSC + cid
        NCG = HD // 16

        def chunk_body(c, carry):
            ch = c * NW + wid

            @pl.when(ch < NCHTOT)
            def _():
                _do_chunk(ch)
            return carry

        def _do_chunk(ch):
            node0 = ch * CH
            pltpu.sync_copy(idx_hbm.at[pl.ds(node0 * K, GR)], ilist)
            pltpu.async_copy(G_hbm.at[ilist], buf.at[pl.ds(0, GR)], sem).wait()
            pltpu.sync_copy(G_hbm.at[pl.ds(node0, CH)], buf.at[pl.ds(GR, CH)])

            def node_body(i, carry2):
                srow = GR + i
                dens, accs, bases = [], [], []
                for cg in range(NCG):
                    bases.append(buf[srow, pl.ds(cg * 16, 16)])
                    dens.append(jnp.ones((16,), jnp.float32))
                    accs.append(buf[srow, pl.ds(HD + cg * 16, 16)])
                for j in range(K):
                    row = i * K + j
                    for cg in range(NCG):
                        sj = buf[row, pl.ds(cg * 16, 16)]
                        uj = buf[row, pl.ds(HD + cg * 16, 16)]
                        e = jnp.exp(bases[cg] - sj)
                        dens[cg] = dens[cg] + e
                        accs[cg] = accs[cg] + e * uj
                for cg in range(NCG):
                    ostage[i, pl.ds(cg * 16, 16)] = accs[cg] / dens[cg]
                return carry2

            lax.fori_loop(0, CH, node_body, 0)
            pltpu.sync_copy(ostage, out_hbm.at[pl.ds(node0, CH)])

        lax.fori_loop(0, NITER, chunk_body, 0)

    return conv(Gm, idxf)


def _mid_body(y_ref, f_ref, wpA, bpA, g0, bt0, wsrc, wlin, wpB, g_ref, sY, qY):
    p_ = pl.program_id(0)
    t_ = pl.program_id(1)
    n = jnp.float32(NT)
    f3 = f_ref[:, 0:3]
    y = (y_ref[...] + jnp.dot(f3, wpA[...], precision=lax.Precision.HIGHEST, preferred_element_type=jnp.float32)
         + bpA[...])

    @pl.when(p_ == 0)
    def _():
        s = jnp.sum(y, axis=0, keepdims=True)
        q = jnp.sum(y * y, axis=0, keepdims=True)
        sY[...] = jnp.where(t_ == 0, s, sY[...] + s)
        qY[...] = jnp.where(t_ == 0, q, qY[...] + q)

    @pl.when(p_ == 1)
    def _():
        mu = sY[...] / n
        var = qY[...] / n - mu * mu
        x = jnp.maximum((y - mu) * lax.rsqrt(var + EPS) * g0[...] + bt0[...],
                        0.0)
        a = jnp.dot(x, wsrc[...], preferred_element_type=jnp.float32)
        v = jnp.dot(x, wlin[...], preferred_element_type=jnp.float32)
        P = jnp.dot(f3, wpB[...], precision=lax.Precision.HIGHEST, preferred_element_type=jnp.float32)
        g_ref[:, 0:HD] = a + P
        g_ref[:, HD:2 * HD] = v - P


def _mid(y, feats, wpA, bpA, g0, bt0, wsrc, wlin, wpB):
    full = lambda s: pl.BlockSpec(s, lambda p, t: (0, 0))
    return pl.pallas_call(
        _mid_body,
        grid=(2, NRT),
        in_specs=[
            pl.BlockSpec((RT, HD), lambda p, t: (t, 0)),
            pl.BlockSpec((RT, 6), lambda p, t: (t, 0)),
            full((3, HD)), full((1, HD)), full((1, HD)), full((1, HD)),
            full((HD, HD)), full((HD, HD)), full((3, HD)),
        ],
        out_specs=pl.BlockSpec((RT, GW), lambda p, t: (t, 0)),
        out_shape=jax.ShapeDtypeStruct((NT, GW), jnp.float32),
        scratch_shapes=[
            pltpu.VMEM((1, HD), jnp.float32),
            pltpu.VMEM((1, HD), jnp.float32),
        ],
    )(y, feats, wpA, bpA, g0, bt0, wsrc, wlin, wpB)


def _final_body(y_ref, f_ref, wpA, bpA, g1, bt1, wd1, bd1, gd, btd, wd2, bd2,
                o_ref, sY, qY, accX, sX):
    p_ = pl.program_id(0)
    t_ = pl.program_id(1)
    n = jnp.float32(NT)
    y = (y_ref[...] + jnp.dot(f_ref[:, 0:3], wpA[...],
                              precision=lax.Precision.HIGHEST, preferred_element_type=jnp.float32) + bpA[...])

    def x_of(y):
        mu = sY[...] / n
        var = qY[...] / n - mu * mu
        return jnp.maximum((y - mu) * lax.rsqrt(var + EPS) * g1[...]
                           + bt1[...], 0.0)

    @pl.when(p_ == 0)
    def _():
        s = jnp.sum(y, axis=0, keepdims=True)
        q = jnp.sum(y * y, axis=0, keepdims=True)
        sY[...] = jnp.where(t_ == 0, s, sY[...] + s)
        qY[...] = jnp.where(t_ == 0, q, qY[...] + q)

    @pl.when(p_ == 1)
    def _():
        x = x_of(y)
        xx = lax.dot_general(x, x, (((0,), (0,)), ((), ())),
                             precision=lax.Precision.HIGHEST, preferred_element_type=jnp.float32)
        sx = jnp.sum(x, axis=0, keepdims=True)
        accX[...] = jnp.where(t_ == 0, xx, accX[...] + xx)
        sX[...] = jnp.where(t_ == 0, sx, sX[...] + sx)

    @pl.when(p_ == 2)
    def _():
        x = x_of(y)
        W = wd1[...]
        ml = jnp.dot(sX[...] / n, W, precision=lax.Precision.HIGHEST, preferred_element_type=jnp.float32)
        MW = jnp.dot(accX[...] / n, W, precision=lax.Precision.HIGHEST, preferred_element_type=jnp.float32)
        var = jnp.sum(MW * W, axis=0, keepdims=True) - ml * ml
        h = jnp.dot(x, W, preferred_element_type=jnp.float32) + bd1[...]
        xd = jnp.maximum((h - (ml + bd1[...])) * lax.rsqrt(var + EPS)
                         * gd[...] + btd[...], 0.0)
        o_ref[...] = jnp.dot(xd, wd2[...],
                             preferred_element_type=jnp.float32) + bd2[...]


def _final(y, feats, wpA, bpA, g1, bt1, wd1, bd1, gd, btd, wd2, bd2):
    full = lambda s: pl.BlockSpec(s, lambda p, t: (0, 0))
    return pl.pallas_call(
        _final_body,
        grid=(3, NRT),
        in_specs=[
            pl.BlockSpec((RT, HD), lambda p, t: (t, 0)),
            pl.BlockSpec((RT, 6), lambda p, t: (t, 0)),
            full((3, HD)), full((1, HD)),
            full((1, HD)), full((1, HD)), full((HD, HD)), full((1, HD)),
            full((1, HD)), full((1, HD)), full((HD, NCLS)), full((1, NCLS)),
        ],
        out_specs=pl.BlockSpec((RT, NCLS), lambda p, t: (t, 0)),
        out_shape=jax.ShapeDtypeStruct((NT, NCLS), jnp.float32),
        scratch_shapes=[
            pltpu.VMEM((1, HD), jnp.float32),
            pltpu.VMEM((1, HD), jnp.float32),
            pltpu.VMEM((HD, HD), jnp.float32),
            pltpu.VMEM((1, HD), jnp.float32),
        ],
    )(y, feats, wpA, bpA, g1, bt1, wd1, bd1, gd, btd, wd2, bd2)


def kernel(features, W_e1, b_e1, g_bn1, bt_bn1, W_e2, b_e2, g_emb, bt_emb,
           W_lin0, W_src0, W_dst0, W_pos0, b_pos0, g_t0, bt_t0,
           W_lin1, W_src1, W_dst1, W_pos1, b_pos1, g_t1, bt_t1,
           W_d1, b_d1, g_d, bt_d, W_d2, b_d2):
    r1 = lambda a: a.reshape(1, -1)
    feats = features.reshape(NT, 6)
    G0 = _embed(feats, W_e1, r1(b_e1), r1(g_bn1), r1(bt_bn1),
                W_e2, r1(b_e2), r1(g_emb), r1(bt_emb), W_src0, W_lin0, W_pos0)
    idxf = _knn(features).reshape(-1)
    y0 = _conv(G0, idxf)
    G1 = _mid(y0, feats, W_pos0, r1(b_pos0), r1(g_t0), r1(bt_t0),
              W_src1, W_lin1, W_pos1)
    y1 = _conv(G1, idxf)
    lg = _final(y1, feats, W_pos1, r1(b_pos1), r1(g_t1), r1(bt_t1),
                W_d1, r1(b_d1), r1(g_d), r1(bt_d), W_d2, r1(b_d2))
    return lg.reshape(B, N, NCLS)


# speedup vs baseline: 12.4600x; 1.6881x over previous
import functools

import jax
import jax.numpy as jnp
from jax import lax
from jax.experimental import pallas as pl
from jax.experimental.pallas import tpu as pltpu
from jax.experimental.pallas import tpu_sc as plsc

B, N, K, NCLS = 4, 5000, 16, 13
ED, HD = 64, 128
NT = B * N
GW = 256
EPS = 1e-5

NSC, NSUB = 2, 16
NW = NSC * NSUB
CH = 8
NCHTOT = NT // CH
NITER = (NCHTOT + NW - 1) // NW
GR = K * CH

RT = 2000
NRT = NT // RT
KT = 200
NKT = N // KT


def _embed_body(f_ref, we1, be1, g1, bt1, we2, be2, g2, bt2, wsrc, wlin, wpos,
                g_ref, accF, sF, accX, sX):
    p_ = pl.program_id(0)
    t_ = pl.program_id(1)
    n = jnp.float32(NT)
    f = f_ref[...]

    def bn_lin(x, W, bvec, g, bt, acc, s):
        mu_x = s[...] / n
        M = acc[...] / n
        ml = jnp.dot(mu_x, W, precision=lax.Precision.HIGHEST, preferred_element_type=jnp.float32)
        MW = jnp.dot(M, W, precision=lax.Precision.HIGHEST, preferred_element_type=jnp.float32)
        var = jnp.sum(MW * W, axis=0, keepdims=True) - ml * ml
        h = jnp.dot(x, W, preferred_element_type=jnp.float32) + bvec
        xo = (h - (ml + bvec)) * lax.rsqrt(var + EPS) * g + bt
        return jnp.maximum(xo, 0.0)

    def x1_of(f):
        return bn_lin(f, we1[...], be1[...], g1[...], bt1[...], accF, sF)

    @pl.when(p_ == 0)
    def _():
        ff = lax.dot_general(f, f, (((0,), (0,)), ((), ())),
                             precision=lax.Precision.HIGHEST, preferred_element_type=jnp.float32)
        sf = jnp.sum(f, axis=0, keepdims=True)
        accF[...] = jnp.where(t_ == 0, ff, accF[...] + ff)
        sF[...] = jnp.where(t_ == 0, sf, sF[...] + sf)

    @pl.when(p_ == 1)
    def _():
        x1 = x1_of(f)
        xx = lax.dot_general(x1, x1, (((0,), (0,)), ((), ())),
                             precision=lax.Precision.HIGHEST, preferred_element_type=jnp.float32)
        sx = jnp.sum(x1, axis=0, keepdims=True)
        accX[...] = jnp.where(t_ == 0, xx, accX[...] + xx)
        sX[...] = jnp.where(t_ == 0, sx, sX[...] + sx)

    @pl.when(p_ == 2)
    def _():
        x1 = x1_of(f)
        x2 = bn_lin(x1, we2[...], be2[...], g2[...], bt2[...], accX, sX)
        a = jnp.dot(x2, wsrc[...], preferred_element_type=jnp.float32)
        v = jnp.dot(x2, wlin[...], preferred_element_type=jnp.float32)
        P = jnp.dot(f[:, 0:3], wpos[...], precision=lax.Precision.HIGHEST, preferred_element_type=jnp.float32)
        g_ref[:, 0:HD] = a + P
        g_ref[:, HD:2 * HD] = v - P


def _embed(feats, we1, be1, g1, bt1, we2, be2, g2, bt2, wsrc, wlin, wpos):
    full = lambda s: pl.BlockSpec(s, lambda p, t: (0, 0))
    return pl.pallas_call(
        _embed_body,
        grid=(3, NRT),
        in_specs=[
            pl.BlockSpec((RT, 6), lambda p, t: (t, 0)),
            full((6, ED)), full((1, ED)), full((1, ED)), full((1, ED)),
            full((ED, ED)), full((1, ED)), full((1, ED)), full((1, ED)),
            full((ED, HD)), full((ED, HD)), full((3, HD)),
        ],
        out_specs=pl.BlockSpec((RT, GW), lambda p, t: (t, 0)),
        out_shape=jax.ShapeDtypeStruct((NT, GW), jnp.float32),
        scratch_shapes=[
            pltpu.VMEM((6, 6), jnp.float32),
            pltpu.VMEM((1, 6), jnp.float32),
            pltpu.VMEM((ED, ED), jnp.float32),
            pltpu.VMEM((1, ED), jnp.float32),
        ],
    )(feats, we1, be1, g1, bt1, we2, be2, g2, bt2, wsrc, wlin, wpos)


def _knn_body(fr_ref, fa_ref, idx_ref):
    b = pl.program_id(0)
    t = pl.program_id(1)
    pr = fr_ref[0, :, 0:3]
    pa = fa_ref[0, :, 0:3]
    sq_r = jnp.sum(pr * pr, axis=1)
    sq_a = jnp.sum(pa * pa, axis=1)
    G = lax.dot_general(pr, pa, (((1,), (1,)), ((), ())),
                        preferred_element_type=jnp.float32)
    d2 = sq_r[:, None] + sq_a[None, :] - 2.0 * G
    colf = lax.broadcasted_iota(jnp.int32, (KT, N), 1).astype(jnp.float32)
    rowg = (jnp.float32(t * KT)
            + lax.broadcasted_iota(jnp.int32, (KT, N), 0).astype(jnp.float32))
    d2 = jnp.where(colf == rowg, 1e30, d2)
    outs = []
    for _ in range(K):
        m = jnp.min(d2, axis=1, keepdims=True)
        sel = jnp.min(jnp.where(d2 == m, colf, jnp.float32(3e9)),
                      axis=1, keepdims=True)
        outs.append(sel)
        d2 = jnp.where(colf == sel, 1e30, d2)
    idx_ref[0, :, :] = (jnp.concatenate(outs, axis=1).astype(jnp.int32)
                        + b * N)


def _knn(features):
    return pl.pallas_call(
        _knn_body,
        grid=(B, NKT),
        in_specs=[
            pl.BlockSpec((1, KT, 6), lambda b, t: (b, t, 0)),
            pl.BlockSpec((1, N, 6), lambda b, t: (b, 0, 0)),
        ],
        out_specs=pl.BlockSpec((1, KT, K), lambda b, t: (b, t, 0)),
        out_shape=jax.ShapeDtypeStruct((B, N, K), jnp.int32),
    )(features, features)


def _conv(Gm, idxf):
    mesh = plsc.VectorSubcoreMesh(core_axis_name="c", subcore_axis_name="s")

    @functools.partial(
        pl.kernel,
        mesh=mesh,
        out_type=jax.ShapeDtypeStruct((NT, HD), jnp.float32),
        scratch_types=[
            pltpu.VMEM((GR + CH, GW), jnp.float32),
            pltpu.VMEM((GR,), jnp.int32),
            pltpu.VMEM((CH, HD), jnp.float32),
            pltpu.SemaphoreType.DMA,
        ],
    )
    def conv(G_hbm, idx_hbm, out_hbm, buf, ilist, ostage, sem):
        cid = lax.axis_index("c")
        sid = lax.axis_index("s")
        wid = sid * NSC + cid
        NCG = HD // 16

        def chunk_body(c, carry):
            ch = c * NW + wid

            @pl.when(ch < NCHTOT)
            def _():
                _do_chunk(ch)
            return carry

        def _do_chunk(ch):
            node0 = ch * CH
            pltpu.sync_copy(idx_hbm.at[pl.ds(node0 * K, GR)], ilist)
            pltpu.async_copy(G_hbm.at[ilist], buf.at[pl.ds(0, GR)], sem).wait()
            pltpu.sync_copy(G_hbm.at[pl.ds(node0, CH)], buf.at[pl.ds(GR, CH)])

            def node_body(i, carry2):
                srow = GR + i
                dens, accs, bases = [], [], []
                for cg in range(NCG):
                    bases.append(buf[srow, pl.ds(cg * 16, 16)])
                    dens.append(jnp.ones((16,), jnp.float32))
                    accs.append(buf[srow, pl.ds(HD + cg * 16, 16)])
                for j in range(K):
                    row = i * K + j
                    for cg in range(NCG):
                        sj = buf[row, pl.ds(cg * 16, 16)]
                        uj = buf[row, pl.ds(HD + cg * 16, 16)]
                        e = jnp.exp(bases[cg] - sj)
                        dens[cg] = dens[cg] + e
                        accs[cg] = accs[cg] + e * uj
                for cg in range(NCG):
                    ostage[i, pl.ds(cg * 16, 16)] = accs[cg] / dens[cg]
                return carry2

            lax.fori_loop(0, CH, node_body, 0)
            pltpu.sync_copy(ostage, out_hbm.at[pl.ds(node0, CH)])

        lax.fori_loop(0, NITER, chunk_body, 0)

    return conv(Gm, idxf)


def _mid_body(y_ref, f_ref, wpA, bpA, g0, bt0, wsrc, wlin, wpB, g_ref, sY, qY):
    p_ = pl.program_id(0)
    t_ = pl.program_id(1)
    n = jnp.float32(NT)
    f3 = f_ref[:, 0:3]
    y = (y_ref[...] + jnp.dot(f3, wpA[...], precision=lax.Precision.HIGHEST, preferred_element_type=jnp.float32)
         + bpA[...])

    @pl.when(p_ == 0)
    def _():
        s = jnp.sum(y, axis=0, keepdims=True)
        q = jnp.sum(y * y, axis=0, keepdims=True)
        sY[...] = jnp.where(t_ == 0, s, sY[...] + s)
        qY[...] = jnp.where(t_ == 0, q, qY[...] + q)

    @pl.when(p_ == 1)
    def _():
        mu = sY[...] / n
        var = qY[...] / n - mu * mu
        x = jnp.maximum((y - mu) * lax.rsqrt(var + EPS) * g0[...] + bt0[...],
                        0.0)
        a = jnp.dot(x, wsrc[...], preferred_element_type=jnp.float32)
        v = jnp.dot(x, wlin[...], preferred_element_type=jnp.float32)
        P = jnp.dot(f3, wpB[...], precision=lax.Precision.HIGHEST, preferred_element_type=jnp.float32)
        g_ref[:, 0:HD] = a + P
        g_ref[:, HD:2 * HD] = v - P


def _mid(y, feats, wpA, bpA, g0, bt0, wsrc, wlin, wpB):
    full = lambda s: pl.BlockSpec(s, lambda p, t: (0, 0))
    return pl.pallas_call(
        _mid_body,
        grid=(2, NRT),
        in_specs=[
            pl.BlockSpec((RT, HD), lambda p, t: (t, 0)),
            pl.BlockSpec((RT, 6), lambda p, t: (t, 0)),
            full((3, HD)), full((1, HD)), full((1, HD)), full((1, HD)),
            full((HD, HD)), full((HD, HD)), full((3, HD)),
        ],
        out_specs=pl.BlockSpec((RT, GW), lambda p, t: (t, 0)),
        out_shape=jax.ShapeDtypeStruct((NT, GW), jnp.float32),
        scratch_shapes=[
            pltpu.VMEM((1, HD), jnp.float32),
            pltpu.VMEM((1, HD), jnp.float32),
        ],
    )(y, feats, wpA, bpA, g0, bt0, wsrc, wlin, wpB)


def _final_body(y_ref, f_ref, wpA, bpA, g1, bt1, wd1, bd1, gd, btd, wd2, bd2,
                o_ref, sY, qY, accX, sX):
    p_ = pl.program_id(0)
    t_ = pl.program_id(1)
    n = jnp.float32(NT)
    y = (y_ref[...] + jnp.dot(f_ref[:, 0:3], wpA[...],
                              precision=lax.Precision.HIGHEST, preferred_element_type=jnp.float32) + bpA[...])

    def x_of(y):
        mu = sY[...] / n
        var = qY[...] / n - mu * mu
        return jnp.maximum((y - mu) * lax.rsqrt(var + EPS) * g1[...]
                           + bt1[...], 0.0)

    @pl.when(p_ == 0)
    def _():
        s = jnp.sum(y, axis=0, keepdims=True)
        q = jnp.sum(y * y, axis=0, keepdims=True)
        sY[...] = jnp.where(t_ == 0, s, sY[...] + s)
        qY[...] = jnp.where(t_ == 0, q, qY[...] + q)

    @pl.when(p_ == 1)
    def _():
        x = x_of(y)
        xx = lax.dot_general(x, x, (((0,), (0,)), ((), ())),
                             precision=lax.Precision.HIGHEST, preferred_element_type=jnp.float32)
        sx = jnp.sum(x, axis=0, keepdims=True)
        accX[...] = jnp.where(t_ == 0, xx, accX[...] + xx)
        sX[...] = jnp.where(t_ == 0, sx, sX[...] + sx)

    @pl.when(p_ == 2)
    def _():
        x = x_of(y)
        W = wd1[...]
        ml = jnp.dot(sX[...] / n, W, precision=lax.Precision.HIGHEST, preferred_element_type=jnp.float32)
        MW = jnp.dot(accX[...] / n, W, precision=lax.Precision.HIGHEST, preferred_element_type=jnp.float32)
        var = jnp.sum(MW * W, axis=0, keepdims=True) - ml * ml
        h = jnp.dot(x, W, preferred_element_type=jnp.float32) + bd1[...]
        xd = jnp.maximum((h - (ml + bd1[...])) * lax.rsqrt(var + EPS)
                         * gd[...] + btd[...], 0.0)
        o_ref[...] = jnp.dot(xd, wd2[...],
                             preferred_element_type=jnp.float32) + bd2[...]


def _final(y, feats, wpA, bpA, g1, bt1, wd1, bd1, gd, btd, wd2, bd2):
    full = lambda s: pl.BlockSpec(s, lambda p, t: (0, 0))
    return pl.pallas_call(
        _final_body,
        grid=(3, NRT),
        in_specs=[
            pl.BlockSpec((RT, HD), lambda p, t: (t, 0)),
            pl.BlockSpec((RT, 6), lambda p, t: (t, 0)),
            full((3, HD)), full((1, HD)),
            full((1, HD)), full((1, HD)), full((HD, HD)), full((1, HD)),
            full((1, HD)), full((1, HD)), full((HD, NCLS)), full((1, NCLS)),
        ],
        out_specs=pl.BlockSpec((RT, NCLS), lambda p, t: (t, 0)),
        out_shape=jax.ShapeDtypeStruct((NT, NCLS), jnp.float32),
        scratch_shapes=[
            pltpu.VMEM((1, HD), jnp.float32),
            pltpu.VMEM((1, HD), jnp.float32),
            pltpu.VMEM((HD, HD), jnp.float32),
            pltpu.VMEM((1, HD), jnp.float32),
        ],
    )(y, feats, wpA, bpA, g1, bt1, wd1, bd1, gd, btd, wd2, bd2)


def kernel(features, W_e1, b_e1, g_bn1, bt_bn1, W_e2, b_e2, g_emb, bt_emb,
           W_lin0, W_src0, W_dst0, W_pos0, b_pos0, g_t0, bt_t0,
           W_lin1, W_src1, W_dst1, W_pos1, b_pos1, g_t1, bt_t1,
           W_d1, b_d1, g_d, bt_d, W_d2, b_d2):
    r1 = lambda a: a.reshape(1, -1)
    feats = features.reshape(NT, 6)
    G0 = _embed(feats, W_e1, r1(b_e1), r1(g_bn1), r1(bt_bn1),
                W_e2, r1(b_e2), r1(g_emb), r1(bt_emb), W_src0, W_lin0, W_pos0)
    idxf = _knn(features).reshape(-1)
    y0 = _conv(G0, idxf)
    G1 = _mid(y0, feats, W_pos0, r1(b_pos0), r1(g_t0), r1(bt_t0),
              W_src1, W_lin1, W_pos1)
    y1 = _conv(G1, idxf)
    lg = _final(y1, feats, W_pos1, r1(b_pos1), r1(g_t1), r1(bt_t1),
                W_d1, r1(b_d1), r1(g_d), r1(bt_d), W_d2, r1(b_d2))
    return lg.reshape(B, N, NCLS)


# speedup vs baseline: 14.8284x; 1.1901x over previous
import functools

import jax
import jax.numpy as jnp
from jax import lax
from jax.experimental import pallas as pl
from jax.experimental.pallas import tpu as pltpu
from jax.experimental.pallas import tpu_sc as plsc

B, N, K, NCLS = 4, 5000, 16, 13
ED, HD = 64, 128
NT = B * N
GW = 256
EPS = 1e-5

NSC, NSUB = 2, 16
NW = NSC * NSUB
CH = 8
NCHTOT = NT // CH
NITER = (NCHTOT + NW - 1) // NW
GR = K * CH

RT = 2000
NRT = NT // RT
KT = 200
NKT = N // KT


def _embed_body(f_ref, we1, be1, g1, bt1, we2, be2, g2, bt2, wsrc, wlin, wpos,
                g_ref, accF, sF, accX, sX):
    p_ = pl.program_id(0)
    t_ = pl.program_id(1)
    n = jnp.float32(NT)
    f = f_ref[...]

    def bn_lin(x, W, bvec, g, bt, acc, s):
        mu_x = s[...] / n
        M = acc[...] / n
        ml = jnp.dot(mu_x, W, precision=lax.Precision.HIGHEST, preferred_element_type=jnp.float32)
        MW = jnp.dot(M, W, precision=lax.Precision.HIGHEST, preferred_element_type=jnp.float32)
        var = jnp.sum(MW * W, axis=0, keepdims=True) - ml * ml
        h = jnp.dot(x, W, preferred_element_type=jnp.float32) + bvec
        xo = (h - (ml + bvec)) * lax.rsqrt(var + EPS) * g + bt
        return jnp.maximum(xo, 0.0)

    def x1_of(f):
        return bn_lin(f, we1[...], be1[...], g1[...], bt1[...], accF, sF)

    @pl.when(p_ == 0)
    def _():
        ff = lax.dot_general(f, f, (((0,), (0,)), ((), ())),
                             precision=lax.Precision.HIGHEST, preferred_element_type=jnp.float32)
        sf = jnp.sum(f, axis=0, keepdims=True)
        accF[...] = jnp.where(t_ == 0, ff, accF[...] + ff)
        sF[...] = jnp.where(t_ == 0, sf, sF[...] + sf)

    @pl.when(p_ == 1)
    def _():
        x1 = x1_of(f)
        xx = lax.dot_general(x1, x1, (((0,), (0,)), ((), ())),
                             precision=lax.Precision.HIGHEST, preferred_element_type=jnp.float32)
        sx = jnp.sum(x1, axis=0, keepdims=True)
        accX[...] = jnp.where(t_ == 0, xx, accX[...] + xx)
        sX[...] = jnp.where(t_ == 0, sx, sX[...] + sx)

    @pl.when(p_ == 2)
    def _():
        x1 = x1_of(f)
        x2 = bn_lin(x1, we2[...], be2[...], g2[...], bt2[...], accX, sX)
        a = jnp.dot(x2, wsrc[...], preferred_element_type=jnp.float32)
        v = jnp.dot(x2, wlin[...], preferred_element_type=jnp.float32)
        P = jnp.dot(f[:, 0:3], wpos[...], precision=lax.Precision.HIGHEST, preferred_element_type=jnp.float32)
        g_ref[:, 0:HD] = a + P
        g_ref[:, HD:2 * HD] = v - P


def _embed(feats, we1, be1, g1, bt1, we2, be2, g2, bt2, wsrc, wlin, wpos):
    full = lambda s: pl.BlockSpec(s, lambda p, t: (0, 0))
    return pl.pallas_call(
        _embed_body,
        grid=(3, NRT),
        in_specs=[
            pl.BlockSpec((RT, 6), lambda p, t: (t, 0)),
            full((6, ED)), full((1, ED)), full((1, ED)), full((1, ED)),
            full((ED, ED)), full((1, ED)), full((1, ED)), full((1, ED)),
            full((ED, HD)), full((ED, HD)), full((3, HD)),
        ],
        out_specs=pl.BlockSpec((RT, GW), lambda p, t: (t, 0)),
        out_shape=jax.ShapeDtypeStruct((NT, GW), jnp.float32),
        scratch_shapes=[
            pltpu.VMEM((6, 6), jnp.float32),
            pltpu.VMEM((1, 6), jnp.float32),
            pltpu.VMEM((ED, ED), jnp.float32),
            pltpu.VMEM((1, ED), jnp.float32),
        ],
    )(feats, we1, be1, g1, bt1, we2, be2, g2, bt2, wsrc, wlin, wpos)


def _knn_body(fr_ref, fa_ref, idx_ref):
    b = pl.program_id(0)
    t = pl.program_id(1)
    pr = fr_ref[0, :, 0:3]
    pa = fa_ref[0, :, 0:3]
    sq_r = jnp.sum(pr * pr, axis=1)
    sq_a = jnp.sum(pa * pa, axis=1)
    G = lax.dot_general(pr, pa, (((1,), (1,)), ((), ())),
                        preferred_element_type=jnp.float32)
    d2 = sq_r[:, None] + sq_a[None, :] - 2.0 * G
    colf = lax.broadcasted_iota(jnp.int32, (KT, N), 1).astype(jnp.float32)
    rowg = (jnp.float32(t * KT)
            + lax.broadcasted_iota(jnp.int32, (KT, N), 0).astype(jnp.float32))
    d2 = jnp.where(colf == rowg, 1e30, d2)
    outs = []
    for _ in range(K):
        m = jnp.min(d2, axis=1, keepdims=True)
        sel = jnp.min(jnp.where(d2 == m, colf, jnp.float32(3e9)),
                      axis=1, keepdims=True)
        outs.append(sel)
        d2 = jnp.where(colf == sel, 1e30, d2)
    idx_ref[0, :, :] = (jnp.concatenate(outs, axis=1).astype(jnp.int32)
                        + b * N)


def _knn(features):
    return pl.pallas_call(
        _knn_body,
        grid=(B, NKT),
        in_specs=[
            pl.BlockSpec((1, KT, 6), lambda b, t: (b, t, 0)),
            pl.BlockSpec((1, N, 6), lambda b, t: (b, 0, 0)),
        ],
        out_specs=pl.BlockSpec((1, KT, K), lambda b, t: (b, t, 0)),
        out_shape=jax.ShapeDtypeStruct((B, N, K), jnp.int32),
    )(features, features)


def _conv(Gm, idxf):
    mesh = plsc.VectorSubcoreMesh(core_axis_name="c", subcore_axis_name="s")

    @functools.partial(
        pl.kernel,
        mesh=mesh,
        out_type=jax.ShapeDtypeStruct((NT, HD), jnp.float32),
        scratch_types=[
            pltpu.VMEM((2, GR + CH, GW), jnp.float32),
            pltpu.VMEM((2, GR), jnp.int32),
            pltpu.VMEM((CH, HD), jnp.float32),
            pltpu.SemaphoreType.DMA,
            pltpu.SemaphoreType.DMA,
        ],
    )
    def conv(G_hbm, idx_hbm, out_hbm, buf, ilist, ostage, sem0, sem1):
        cid = lax.axis_index("c")
        sid = lax.axis_index("s")
        wid = sid * NSC + cid
        NCG = HD // 16
        sems = (sem0, sem1)

        def issue(t, b):
            ch = t * NW + wid

            @pl.when((t < NITER) & (ch < NCHTOT))
            def _():
                node0 = ch * CH
                pltpu.sync_copy(idx_hbm.at[pl.ds(node0 * K, GR)],
                                ilist.at[b])
                pltpu.async_copy(G_hbm.at[ilist.at[b]],
                                 buf.at[b, pl.ds(0, GR)], sems[b])
                pltpu.async_copy(G_hbm.at[pl.ds(node0, CH)],
                                 buf.at[b, pl.ds(GR, CH)], sems[b])

        def consume(t, b):
            ch = t * NW + wid

            @pl.when((t < NITER) & (ch < NCHTOT))
            def _():
                node0 = ch * CH
                pltpu.make_async_copy(G_hbm.at[pl.ds(0, GR)],
                                      buf.at[b, pl.ds(0, GR)],
                                      sems[b]).wait()
                pltpu.make_async_copy(G_hbm.at[pl.ds(0, CH)],
                                      buf.at[b, pl.ds(GR, CH)],
                                      sems[b]).wait()

                def node_body(i, carry2):
                    srow = GR + i
                    dens, accs, bases = [], [], []
                    for cg in range(NCG):
                        bases.append(buf[b, srow, pl.ds(cg * 16, 16)])
                        dens.append(jnp.ones((16,), jnp.float32))
                        accs.append(buf[b, srow, pl.ds(HD + cg * 16, 16)])
                    for j in range(K):
                        row = i * K + j
                        for cg in range(NCG):
                            sj = buf[b, row, pl.ds(cg * 16, 16)]
                            uj = buf[b, row, pl.ds(HD + cg * 16, 16)]
                            e = jnp.exp(bases[cg] - sj)
                            dens[cg] = dens[cg] + e
                            accs[cg] = accs[cg] + e * uj
                    for cg in range(NCG):
                        ostage[i, pl.ds(cg * 16, 16)] = accs[cg] / dens[cg]
                    return carry2

                lax.fori_loop(0, CH, node_body, 0)
                pltpu.sync_copy(ostage, out_hbm.at[pl.ds(node0, CH)])

        issue(0, 0)

        def outer(t2, carry):
            for bb in range(2):
                t = 2 * t2 + bb
                issue(t + 1, 1 - bb)
                consume(t, bb)
            return carry

        lax.fori_loop(0, (NITER + 1) // 2, outer, 0)

    return conv(Gm, idxf)


def _mid_body(y_ref, f_ref, wpA, bpA, g0, bt0, wsrc, wlin, wpB, g_ref, sY, qY):
    p_ = pl.program_id(0)
    t_ = pl.program_id(1)
    n = jnp.float32(NT)
    f3 = f_ref[:, 0:3]
    y = (y_ref[...] + jnp.dot(f3, wpA[...], precision=lax.Precision.HIGHEST, preferred_element_type=jnp.float32)
         + bpA[...])

    @pl.when(p_ == 0)
    def _():
        s = jnp.sum(y, axis=0, keepdims=True)
        q = jnp.sum(y * y, axis=0, keepdims=True)
        sY[...] = jnp.where(t_ == 0, s, sY[...] + s)
        qY[...] = jnp.where(t_ == 0, q, qY[...] + q)

    @pl.when(p_ == 1)
    def _():
        mu = sY[...] / n
        var = qY[...] / n - mu * mu
        x = jnp.maximum((y - mu) * lax.rsqrt(var + EPS) * g0[...] + bt0[...],
                        0.0)
        a = jnp.dot(x, wsrc[...], preferred_element_type=jnp.float32)
        v = jnp.dot(x, wlin[...], preferred_element_type=jnp.float32)
        P = jnp.dot(f3, wpB[...], precision=lax.Precision.HIGHEST, preferred_element_type=jnp.float32)
        g_ref[:, 0:HD] = a + P
        g_ref[:, HD:2 * HD] = v - P


def _mid(y, feats, wpA, bpA, g0, bt0, wsrc, wlin, wpB):
    full = lambda s: pl.BlockSpec(s, lambda p, t: (0, 0))
    return pl.pallas_call(
        _mid_body,
        grid=(2, NRT),
        in_specs=[
            pl.BlockSpec((RT, HD), lambda p, t: (t, 0)),
            pl.BlockSpec((RT, 6), lambda p, t: (t, 0)),
            full((3, HD)), full((1, HD)), full((1, HD)), full((1, HD)),
            full((HD, HD)), full((HD, HD)), full((3, HD)),
        ],
        out_specs=pl.BlockSpec((RT, GW), lambda p, t: (t, 0)),
        out_shape=jax.ShapeDtypeStruct((NT, GW), jnp.float32),
        scratch_shapes=[
            pltpu.VMEM((1, HD), jnp.float32),
            pltpu.VMEM((1, HD), jnp.float32),
        ],
    )(y, feats, wpA, bpA, g0, bt0, wsrc, wlin, wpB)


def _final_body(y_ref, f_ref, wpA, bpA, g1, bt1, wd1, bd1, gd, btd, wd2, bd2,
                o_ref, sY, qY, accX, sX):
    p_ = pl.program_id(0)
    t_ = pl.program_id(1)
    n = jnp.float32(NT)
    y = (y_ref[...] + jnp.dot(f_ref[:, 0:3], wpA[...],
                              precision=lax.Precision.HIGHEST, preferred_element_type=jnp.float32) + bpA[...])

    def x_of(y):
        mu = sY[...] / n
        var = qY[...] / n - mu * mu
        return jnp.maximum((y - mu) * lax.rsqrt(var + EPS) * g1[...]
                           + bt1[...], 0.0)

    @pl.when(p_ == 0)
    def _():
        s = jnp.sum(y, axis=0, keepdims=True)
        q = jnp.sum(y * y, axis=0, keepdims=True)
        sY[...] = jnp.where(t_ == 0, s, sY[...] + s)
        qY[...] = jnp.where(t_ == 0, q, qY[...] + q)

    @pl.when(p_ == 1)
    def _():
        x = x_of(y)
        xx = lax.dot_general(x, x, (((0,), (0,)), ((), ())),
                             precision=lax.Precision.HIGHEST, preferred_element_type=jnp.float32)
        sx = jnp.sum(x, axis=0, keepdims=True)
        accX[...] = jnp.where(t_ == 0, xx, accX[...] + xx)
        sX[...] = jnp.where(t_ == 0, sx, sX[...] + sx)

    @pl.when(p_ == 2)
    def _():
        x = x_of(y)
        W = wd1[...]
        ml = jnp.dot(sX[...] / n, W, precision=lax.Precision.HIGHEST, preferred_element_type=jnp.float32)
        MW = jnp.dot(accX[...] / n, W, precision=lax.Precision.HIGHEST, preferred_element_type=jnp.float32)
        var = jnp.sum(MW * W, axis=0, keepdims=True) - ml * ml
        h = jnp.dot(x, W, preferred_element_type=jnp.float32) + bd1[...]
        xd = jnp.maximum((h - (ml + bd1[...])) * lax.rsqrt(var + EPS)
                         * gd[...] + btd[...], 0.0)
        o_ref[...] = jnp.dot(xd, wd2[...],
                             preferred_element_type=jnp.float32) + bd2[...]


def _final(y, feats, wpA, bpA, g1, bt1, wd1, bd1, gd, btd, wd2, bd2):
    full = lambda s: pl.BlockSpec(s, lambda p, t: (0, 0))
    return pl.pallas_call(
        _final_body,
        grid=(3, NRT),
        in_specs=[
            pl.BlockSpec((RT, HD), lambda p, t: (t, 0)),
            pl.BlockSpec((RT, 6), lambda p, t: (t, 0)),
            full((3, HD)), full((1, HD)),
            full((1, HD)), full((1, HD)), full((HD, HD)), full((1, HD)),
            full((1, HD)), full((1, HD)), full((HD, NCLS)), full((1, NCLS)),
        ],
        out_specs=pl.BlockSpec((RT, NCLS), lambda p, t: (t, 0)),
        out_shape=jax.ShapeDtypeStruct((NT, NCLS), jnp.float32),
        scratch_shapes=[
            pltpu.VMEM((1, HD), jnp.float32),
            pltpu.VMEM((1, HD), jnp.float32),
            pltpu.VMEM((HD, HD), jnp.float32),
            pltpu.VMEM((1, HD), jnp.float32),
        ],
    )(y, feats, wpA, bpA, g1, bt1, wd1, bd1, gd, btd, wd2, bd2)


def kernel(features, W_e1, b_e1, g_bn1, bt_bn1, W_e2, b_e2, g_emb, bt_emb,
           W_lin0, W_src0, W_dst0, W_pos0, b_pos0, g_t0, bt_t0,
           W_lin1, W_src1, W_dst1, W_pos1, b_pos1, g_t1, bt_t1,
           W_d1, b_d1, g_d, bt_d, W_d2, b_d2):
    r1 = lambda a: a.reshape(1, -1)
    feats = features.reshape(NT, 6)
    G0 = _embed(feats, W_e1, r1(b_e1), r1(g_bn1), r1(bt_bn1),
                W_e2, r1(b_e2), r1(g_emb), r1(bt_emb), W_src0, W_lin0, W_pos0)
    idxf = _knn(features).reshape(-1)
    y0 = _conv(G0, idxf)
    G1 = _mid(y0, feats, W_pos0, r1(b_pos0), r1(g_t0), r1(bt_t0),
              W_src1, W_lin1, W_pos1)
    y1 = _conv(G1, idxf)
    lg = _final(y1, feats, W_pos1, r1(b_pos1), r1(g_t1), r1(bt_t1),
                W_d1, r1(b_d1), r1(g_d), r1(bt_d), W_d2, r1(b_d2))
    return lg.reshape(B, N, NCLS)
